# baseline (speedup 1.0000x reference)
import jax
import jax.numpy as jnp
from jax import lax
from jax.experimental import pallas as pl
from jax.experimental.pallas import tpu as pltpu
from jax.experimental.pallas import tpu_sc as plsc

N = 10000
E = 320000
D = 128
DH = D // 2

NC = 2
NS = 16

C = 128
CHUNKS = 160
E_PAD = NS * CHUNKS * C
N_PAD = 10240
ROWS_PER_TILE = N_PAD // NS
NBUF = 2


def _mm_body(x_ref, w_ref, o_ref):
    o_ref[0] = jnp.dot(x_ref[...], w_ref[0],
                       preferred_element_type=jnp.float32)


def _matmul_halves(x, w):
    bm = 1000
    w2 = w.reshape(D, NC, DH).transpose(1, 0, 2)
    return pl.pallas_call(
        _mm_body,
        grid=(N // bm, NC),
        in_specs=[
            pl.BlockSpec((bm, D), lambda i, p: (i, 0)),
            pl.BlockSpec((1, D, DH), lambda i, p: (p, 0, 0)),
        ],
        out_specs=pl.BlockSpec((1, bm, DH), lambda i, p: (p, i, 0)),
        out_shape=jax.ShapeDtypeStruct((NC, N_PAD, DH), jnp.float32),
    )(x, w2)


def _sc_body(support_hbm, packed_hbm, adj_hbm, binit_hbm, out_hbm,
             table, acc, packed_v, col_s, row_s, adj_s, rows_b,
             semg, sems, sema):
    c = lax.axis_index("c")
    s = lax.axis_index("s")
    rslice = pl.ds(s * ROWS_PER_TILE, ROWS_PER_TILE)

    pltpu.sync_copy(support_hbm.at[c, rslice], table.at[rslice])
    pltpu.sync_copy(binit_hbm.at[c, rslice], acc.at[rslice])

    pltpu.sync_copy(packed_hbm.at[s], packed_v)
    plsc.subcore_barrier()

    def fill_slot(g, m):
        for j in range(C // 16):
            sl = pl.ds(j * 16, 16)
            p = packed_v[g, sl]
            col_s[m][sl] = p & 32767
            row_s[m][sl] = p >> 15

    def start_adj(g, m):
        pltpu.async_copy(adj_hbm.at[s, g], adj_s[m], sema[m])

    def wait_adj(m):
        pltpu.make_async_copy(adj_hbm.at[s, 0], adj_s[m], sema[m]).wait()

    def start_gather(m):
        pltpu.async_copy(table.at[col_s[m]], rows_b[m], semg[m])

    def wait_gather(m):
        pltpu.make_async_copy(table.at[col_s[m]], rows_b[m], semg[m]).wait()

    def start_scatter(m):
        pltpu.async_copy(rows_b[m], acc.at[row_s[m]], sems[m], add=True)

    def wait_scatter(m):
        pltpu.make_async_copy(rows_b[m], acc.at[row_s[m]], sems[m]).wait()

    def scale(m):
        rows_v = rows_b[m]
        adj_v = adj_s[m]

        def scale_body(it, carry2):
            a16 = adj_v[pl.ds(it * 16, 16)]
            for k in range(16):
                e = it * 16 + k
                a = a16[k]
                for j in range(DH // 16):
                    sl = pl.ds(j * 16, 16)
                    rows_v[e, sl] = rows_v[e, sl] * a
            return carry2

        lax.fori_loop(0, C // 16, scale_body, 0)

    for m in range(NBUF):
        fill_slot(m, m)
        start_adj(m, m)
        start_gather(m)

    def body(i, carry):
        for m in range(NBUF):
            g = i * NBUF + m
            wait_gather(m)
            wait_adj(m)
            scale(m)
            start_scatter(m)

            @pl.when(g + NBUF < CHUNKS)
            def _():
                wait_scatter(m)
                fill_slot(g + NBUF, m)
                start_adj(g + NBUF, m)
                start_gather(m)

        return carry

    lax.fori_loop(0, CHUNKS // NBUF, body, 0)
    for m in range(NBUF):
        wait_scatter(m)

    plsc.subcore_barrier()
    pltpu.sync_copy(acc.at[rslice], out_hbm.at[c, rslice])


def _sc_edge_pass(support2, packed, adj, binit):
    mesh = plsc.VectorSubcoreMesh(core_axis_name="c", subcore_axis_name="s",
                                  num_cores=NC, num_subcores=NS)
    k = pl.kernel(
        _sc_body,
        out_type=jax.ShapeDtypeStruct((NC, N_PAD, DH), jnp.float32),
        mesh=mesh,
        compiler_params=pltpu.CompilerParams(use_tc_tiling_on_sc=False),
        scratch_types=[
            pltpu.VMEM_SHARED((N_PAD, DH), jnp.float32),
            pltpu.VMEM_SHARED((N_PAD, DH), jnp.float32),
            pltpu.VMEM((CHUNKS, C), jnp.int32),
            [pltpu.VMEM((C,), jnp.int32)] * NBUF,
            [pltpu.VMEM((C,), jnp.int32)] * NBUF,
            [pltpu.VMEM((C,), jnp.float32)] * NBUF,
            [pltpu.VMEM((C, DH), jnp.float32)] * NBUF,
            [pltpu.SemaphoreType.DMA] * NBUF,
            [pltpu.SemaphoreType.DMA] * NBUF,
            [pltpu.SemaphoreType.DMA] * NBUF,
        ],
    )
    return k(support2,
             packed.reshape(NS, CHUNKS, C),
             adj.reshape(NS, CHUNKS, C), binit)


def _comb_body(p_ref, o_ref):
    o_ref[...] = jnp.concatenate([p_ref[0], p_ref[1]], axis=1)


def _combine(partials):
    bm = 1000
    return pl.pallas_call(
        _comb_body,
        grid=(N // bm,),
        in_specs=[
            pl.BlockSpec((NC, bm, DH), lambda i: (0, i, 0)),
        ],
        out_specs=pl.BlockSpec((bm, D), lambda i: (i, 0)),
        out_shape=jax.ShapeDtypeStruct((N, D), jnp.float32),
    )(partials)


def kernel(x, edge_index, adj_values, W0, b0):
    support2 = _matmul_halves(x, W0)

    pad = E_PAD - E
    row = jnp.concatenate(
        [edge_index[0], jnp.zeros((pad,), dtype=jnp.int32)])
    col = jnp.concatenate(
        [edge_index[1], jnp.zeros((pad,), dtype=jnp.int32)])
    adj = jnp.concatenate(
        [adj_values, jnp.zeros((pad,), dtype=jnp.float32)])
    packed = (row << 15) | col
    binit = jnp.broadcast_to(
        b0.reshape(NC, 1, DH), (NC, N_PAD, DH))

    partials = _sc_edge_pass(support2, packed, adj, binit)
    return _combine(partials)


# speedup vs baseline: 1.0215x; 1.0215x over previous
import jax
import jax.numpy as jnp
from jax import lax
from jax.experimental import pallas as pl
from jax.experimental.pallas import tpu as pltpu
from jax.experimental.pallas import tpu_sc as plsc

N = 10000
E = 320000
D = 128
DH = D // 2

NC = 2
NS = 16

C = 128
CHUNKS = 160
E_PAD = NS * CHUNKS * C
N_PAD = 10240
ROWS_PER_TILE = N_PAD // NS
NBUF = 2


def _mm_body(x_ref, w_ref, o_ref):
    o_ref[0] = jnp.dot(x_ref[...], w_ref[0],
                       preferred_element_type=jnp.float32)
    o_ref[1] = jnp.dot(x_ref[...], w_ref[1],
                       preferred_element_type=jnp.float32)


def _matmul_halves(x, w):
    bm = 2000
    w2 = w.reshape(D, NC, DH).transpose(1, 0, 2)
    return pl.pallas_call(
        _mm_body,
        grid=(N // bm,),
        in_specs=[
            pl.BlockSpec((bm, D), lambda i: (i, 0)),
            pl.BlockSpec((NC, D, DH), lambda i: (0, 0, 0)),
        ],
        out_specs=pl.BlockSpec((NC, bm, DH), lambda i: (0, i, 0)),
        out_shape=jax.ShapeDtypeStruct((NC, N_PAD, DH), jnp.float32),
    )(x, w2)


def _sc_body(support_hbm, packed_hbm, adj_hbm, binit_hbm, out_hbm,
             table, acc, packed_v, col_s, row_s, adj_s, rows_b,
             semg, sems, sema):
    c = lax.axis_index("c")
    s = lax.axis_index("s")
    rslice = pl.ds(s * ROWS_PER_TILE, ROWS_PER_TILE)

    pltpu.sync_copy(support_hbm.at[c, rslice], table.at[rslice])
    pltpu.sync_copy(binit_hbm.at[c, rslice], acc.at[rslice])

    pltpu.sync_copy(packed_hbm.at[s], packed_v)
    plsc.subcore_barrier()

    def fill_slot(g, m):
        for j in range(C // 16):
            sl = pl.ds(j * 16, 16)
            p = packed_v[g, sl]
            col_s[m][sl] = p & 32767
            row_s[m][sl] = p >> 15

    def start_adj(g, m):
        pltpu.async_copy(adj_hbm.at[s, g], adj_s[m], sema[m])

    def wait_adj(m):
        pltpu.make_async_copy(adj_hbm.at[s, 0], adj_s[m], sema[m]).wait()

    def start_gather(m):
        pltpu.async_copy(table.at[col_s[m]], rows_b[m], semg[m])

    def wait_gather(m):
        pltpu.make_async_copy(table.at[col_s[m]], rows_b[m], semg[m]).wait()

    def start_scatter(m):
        pltpu.async_copy(rows_b[m], acc.at[row_s[m]], sems[m], add=True)

    def wait_scatter(m):
        pltpu.make_async_copy(rows_b[m], acc.at[row_s[m]], sems[m]).wait()

    def scale(m):
        rows_v = rows_b[m]
        adj_v = adj_s[m]

        def scale_body(it, carry2):
            a16 = adj_v[pl.ds(it * 16, 16)]
            for k in range(16):
                e = it * 16 + k
                a = a16[k]
                for j in range(DH // 16):
                    sl = pl.ds(j * 16, 16)
                    rows_v[e, sl] = rows_v[e, sl] * a
            return carry2

        lax.fori_loop(0, C // 16, scale_body, 0)

    for m in range(NBUF):
        fill_slot(m, m)
        start_adj(m, m)
        start_gather(m)

    def body(i, carry):
        for m in range(NBUF):
            g = i * NBUF + m
            wait_gather(m)
            wait_adj(m)
            scale(m)
            start_scatter(m)

            @pl.when(g + NBUF < CHUNKS)
            def _():
                wait_scatter(m)
                fill_slot(g + NBUF, m)
                start_adj(g + NBUF, m)
                start_gather(m)

        return carry

    lax.fori_loop(0, CHUNKS // NBUF, body, 0)
    for m in range(NBUF):
        wait_scatter(m)

    plsc.subcore_barrier()
    pltpu.sync_copy(acc.at[rslice], out_hbm.at[c, rslice])


def _sc_edge_pass(support2, packed, adj, binit):
    mesh = plsc.VectorSubcoreMesh(core_axis_name="c", subcore_axis_name="s",
                                  num_cores=NC, num_subcores=NS)
    k = pl.kernel(
        _sc_body,
        out_type=jax.ShapeDtypeStruct((NC, N_PAD, DH), jnp.float32),
        mesh=mesh,
        compiler_params=pltpu.CompilerParams(use_tc_tiling_on_sc=False),
        scratch_types=[
            pltpu.VMEM_SHARED((N_PAD, DH), jnp.float32),
            pltpu.VMEM_SHARED((N_PAD, DH), jnp.float32),
            pltpu.VMEM((CHUNKS, C), jnp.int32),
            [pltpu.VMEM((C,), jnp.int32)] * NBUF,
            [pltpu.VMEM((C,), jnp.int32)] * NBUF,
            [pltpu.VMEM((C,), jnp.float32)] * NBUF,
            [pltpu.VMEM((C, DH), jnp.float32)] * NBUF,
            [pltpu.SemaphoreType.DMA] * NBUF,
            [pltpu.SemaphoreType.DMA] * NBUF,
            [pltpu.SemaphoreType.DMA] * NBUF,
        ],
    )
    return k(support2,
             packed.reshape(NS, CHUNKS, C),
             adj.reshape(NS, CHUNKS, C), binit)


def _comb_body(p_ref, o_ref):
    o_ref[...] = jnp.concatenate([p_ref[0], p_ref[1]], axis=1)


def _combine(partials):
    bm = 1000
    return pl.pallas_call(
        _comb_body,
        grid=(N // bm,),
        in_specs=[
            pl.BlockSpec((NC, bm, DH), lambda i: (0, i, 0)),
        ],
        out_specs=pl.BlockSpec((bm, D), lambda i: (i, 0)),
        out_shape=jax.ShapeDtypeStruct((N, D), jnp.float32),
    )(partials)


def kernel(x, edge_index, adj_values, W0, b0):
    support2 = _matmul_halves(x, W0)

    pad = E_PAD - E
    ef = edge_index.reshape(2 * E)
    packed = jnp.concatenate(
        [(ef[:E] << 15) | ef[E:], jnp.zeros((pad,), dtype=jnp.int32)])
    adj = jnp.concatenate(
        [adj_values, jnp.zeros((pad,), dtype=jnp.float32)])
    binit = jnp.broadcast_to(
        b0.reshape(NC, 1, DH), (NC, N_PAD, DH))

    partials = _sc_edge_pass(support2, packed, adj, binit)
    return _combine(partials)
